# baseline (speedup 1.0000x reference)
import jax
import jax.numpy as jnp
from jax.experimental import pallas as pl

BLOCK_S = 512


def _add_body(x_ref, e_ref, o_ref):
    o_ref[0] = x_ref[0] + e_ref[...]


def kernel(inputs, embeddings):
    b, s, d = inputs.shape
    emb = embeddings[:s]
    grid = (s // BLOCK_S, b)
    return pl.pallas_call(
        _add_body,
        grid=grid,
        in_specs=[
            pl.BlockSpec((1, BLOCK_S, d), lambda i, j: (j, i, 0)),
            pl.BlockSpec((BLOCK_S, d), lambda i, j: (i, 0)),
        ],
        out_specs=pl.BlockSpec((1, BLOCK_S, d), lambda i, j: (j, i, 0)),
        out_shape=jax.ShapeDtypeStruct((b, s, d), inputs.dtype),
    )(inputs, emb)


# speedup vs baseline: 1.1147x; 1.1147x over previous
import jax
import jax.numpy as jnp
from jax.experimental import pallas as pl

BLOCK_S = 1024


def _add_body(x_ref, e_ref, o_ref):
    o_ref[0] = x_ref[0] + e_ref[...]


def kernel(inputs, embeddings):
    b, s, d = inputs.shape
    emb = embeddings[:s]
    grid = (s // BLOCK_S, b)
    return pl.pallas_call(
        _add_body,
        grid=grid,
        in_specs=[
            pl.BlockSpec((1, BLOCK_S, d), lambda i, j: (j, i, 0)),
            pl.BlockSpec((BLOCK_S, d), lambda i, j: (i, 0)),
        ],
        out_specs=pl.BlockSpec((1, BLOCK_S, d), lambda i, j: (j, i, 0)),
        out_shape=jax.ShapeDtypeStruct((b, s, d), inputs.dtype),
    )(inputs, emb)


# speedup vs baseline: 1.1613x; 1.0418x over previous
import jax
import jax.numpy as jnp
from jax.experimental import pallas as pl

BLOCK_S = 2048


def _add_body(x_ref, e_ref, o_ref):
    o_ref[0] = x_ref[0] + e_ref[...]


def kernel(inputs, embeddings):
    b, s, d = inputs.shape
    emb = embeddings[:s]
    grid = (s // BLOCK_S, b)
    return pl.pallas_call(
        _add_body,
        grid=grid,
        in_specs=[
            pl.BlockSpec((1, BLOCK_S, d), lambda i, j: (j, i, 0)),
            pl.BlockSpec((BLOCK_S, d), lambda i, j: (i, 0)),
        ],
        out_specs=pl.BlockSpec((1, BLOCK_S, d), lambda i, j: (j, i, 0)),
        out_shape=jax.ShapeDtypeStruct((b, s, d), inputs.dtype),
    )(inputs, emb)
